# baseline (speedup 1.0000x reference)
import functools

import jax
import jax.numpy as jnp
from jax import lax
from jax.experimental import pallas as pl
from jax.experimental.pallas import tpu as pltpu
from jax.experimental.pallas import tpu_sc as plsc

D_MODEL = 768
L1 = 2048
L2 = 2048
BATCH = 16
NCOORD = 5
LANES = BATCH * NCOORD
H = 256
LTOT = L1 + L2
NUM_CHUNKS = LTOT // H
EXT = L2 + LTOT
DOUT = BATCH * D_MODEL
HALF = LTOT // 2


def _tc_body(lens1_ref, b1_ref, b2_ref, l1lane_ref, wbd_ref, out_ref):
    i = pl.program_id(0)
    j0 = i * H
    rowid = j0 + jax.lax.broadcasted_iota(jnp.int32, (H, 1), 0)
    laneq = jax.lax.broadcasted_iota(jnp.int32, (H, LANES), 1) // NCOORD
    src2 = jnp.zeros((H, LANES), jnp.float32)
    for col in range(BATCH):
        start = L2 + j0 - lens1_ref[col]
        win = b2_ref[pl.ds(start, H), :]
        src2 = jnp.where(laneq == col, win, src2)
    src = jnp.where(rowid < l1lane_ref[...], b1_ref[...], src2)
    res = jnp.dot(src, wbd_ref[...], preferred_element_type=jnp.float32)
    out_ref[...] = res.reshape(H, BATCH, D_MODEL)


def _sc_mask_body(l1_hbm, l2_hbm, out_hbm, l1v, l2v, flen2, row_v):
    wid = lax.axis_index("s") * 2 + lax.axis_index("c")
    b = wid // 2
    p0 = (wid % 2) * HALF
    pltpu.sync_copy(l1_hbm, l1v)
    pltpu.sync_copy(l2_hbm, l2v)
    lane = lax.broadcasted_iota(jnp.int32, (16,), 0)
    flen = l1v[...] + l2v[...]
    flen2[pl.ds(0, 16)] = flen
    flen2[pl.ds(16, 16)] = flen
    fb = flen2[pl.ds(b, 16)][0]
    for chunk in range(HALF // 16):
        pos = p0 + chunk * 16 + lane
        row_v[pl.ds(chunk * 16, 16)] = jnp.where(pos >= fb, 1, 0)
    pltpu.sync_copy(row_v, out_hbm.at[pl.ds(wid * HALF, HALF)])


def kernel(boxes1, lengths1, boxes2, lengths2, W, b):
    ones1 = jnp.ones((L1, BATCH, 1), jnp.float32)
    b1_flat = jnp.concatenate([boxes1, ones1], axis=2).reshape(L1, LANES)
    b2a = jnp.concatenate([boxes2, ones1], axis=2).reshape(L2, LANES)
    b2_flat = jnp.pad(b2a, ((L2, EXT - L2 - L2), (0, 0)))
    l1lane = jnp.repeat(lengths1, NCOORD).reshape(1, LANES)
    w5 = jnp.concatenate([W, b.reshape(1, D_MODEL)], axis=0)
    wbd = jnp.kron(jnp.eye(BATCH, dtype=W.dtype), w5)

    grid_spec = pltpu.PrefetchScalarGridSpec(
        num_scalar_prefetch=1,
        grid=(NUM_CHUNKS,),
        in_specs=[
            pl.BlockSpec((H, LANES), lambda i, s: (i, 0)),
            pl.BlockSpec((EXT, LANES), lambda i, s: (0, 0)),
            pl.BlockSpec((1, LANES), lambda i, s: (0, 0)),
            pl.BlockSpec((LANES, DOUT), lambda i, s: (0, 0)),
        ],
        out_specs=pl.BlockSpec((H, BATCH, D_MODEL), lambda i, s: (i, 0, 0)),
    )
    out = pl.pallas_call(
        _tc_body,
        grid_spec=grid_spec,
        out_shape=jax.ShapeDtypeStruct((LTOT, BATCH, D_MODEL), jnp.float32),
        compiler_params=pltpu.CompilerParams(
            dimension_semantics=("arbitrary",),
        ),
    )(lengths1, b1_flat, b2_flat, l1lane, wbd)

    mask_i32 = functools.partial(
        pl.kernel,
        out_type=jax.ShapeDtypeStruct((BATCH * LTOT,), jnp.int32),
        mesh=plsc.VectorSubcoreMesh(core_axis_name="c", subcore_axis_name="s"),
        scratch_types=[
            pltpu.VMEM((BATCH,), jnp.int32),
            pltpu.VMEM((BATCH,), jnp.int32),
            pltpu.VMEM((2 * BATCH,), jnp.int32),
            pltpu.VMEM((HALF,), jnp.int32),
        ],
    )(_sc_mask_body)(lengths1, lengths2)
    return out, mask_i32.reshape(BATCH, LTOT).astype(jnp.bool_)


# speedup vs baseline: 1.1283x; 1.1283x over previous
import jax
import jax.numpy as jnp
from jax.experimental import pallas as pl
from jax.experimental.pallas import tpu as pltpu

D_MODEL = 768
L1 = 2048
L2 = 2048
BATCH = 16
NCOORD = 5
LANES = BATCH * NCOORD
H = 256
LTOT = L1 + L2
NUM_CHUNKS = LTOT // H
EXT = L2 + LTOT
DOUT = BATCH * D_MODEL


def _body(lens1_ref, b1_ref, b2_ref, l1lane_ref, l1c_ref, l2c_ref,
          wbd_ref, out_ref, mask_ref):
    i = pl.program_id(0)
    j0 = i * H
    rowid = j0 + jax.lax.broadcasted_iota(jnp.int32, (H, 1), 0)
    laneq = jax.lax.broadcasted_iota(jnp.int32, (H, LANES), 1) // NCOORD
    src2 = jnp.zeros((H, LANES), jnp.float32)
    for col in range(BATCH):
        start = L2 + j0 - lens1_ref[col]
        win = b2_ref[pl.ds(start, H), :]
        src2 = jnp.where(laneq == col, win, src2)
    src = jnp.where(rowid < l1lane_ref[...], b1_ref[...], src2)
    res = jnp.dot(src, wbd_ref[...], preferred_element_type=jnp.float32)
    out_ref[...] = res.reshape(H, BATCH, D_MODEL)
    flens = l1c_ref[...] + l2c_ref[...]
    colid = j0 + jax.lax.broadcasted_iota(jnp.int32, (BATCH, H), 1)
    mask_ref[...] = colid >= flens


def kernel(boxes1, lengths1, boxes2, lengths2, W, b):
    ones1 = jnp.ones((L1, BATCH, 1), jnp.float32)
    b1_flat = jnp.concatenate([boxes1, ones1], axis=2).reshape(L1, LANES)
    b2a = jnp.concatenate([boxes2, ones1], axis=2).reshape(L2, LANES)
    b2_flat = jnp.pad(b2a, ((L2, EXT - L2 - L2), (0, 0)))
    l1lane = jnp.repeat(lengths1, NCOORD).reshape(1, LANES)
    l1c = lengths1.reshape(BATCH, 1)
    l2c = lengths2.reshape(BATCH, 1)
    w5 = jnp.concatenate([W, b.reshape(1, D_MODEL)], axis=0)
    wbd = jnp.kron(jnp.eye(BATCH, dtype=W.dtype), w5)

    grid_spec = pltpu.PrefetchScalarGridSpec(
        num_scalar_prefetch=1,
        grid=(NUM_CHUNKS,),
        in_specs=[
            pl.BlockSpec((H, LANES), lambda i, s: (i, 0)),
            pl.BlockSpec((EXT, LANES), lambda i, s: (0, 0)),
            pl.BlockSpec((1, LANES), lambda i, s: (0, 0)),
            pl.BlockSpec((BATCH, 1), lambda i, s: (0, 0)),
            pl.BlockSpec((BATCH, 1), lambda i, s: (0, 0)),
            pl.BlockSpec((LANES, DOUT), lambda i, s: (0, 0)),
        ],
        out_specs=[
            pl.BlockSpec((H, BATCH, D_MODEL), lambda i, s: (i, 0, 0)),
            pl.BlockSpec((BATCH, H), lambda i, s: (0, i)),
        ],
    )
    out, mask = pl.pallas_call(
        _body,
        grid_spec=grid_spec,
        out_shape=[
            jax.ShapeDtypeStruct((LTOT, BATCH, D_MODEL), jnp.float32),
            jax.ShapeDtypeStruct((BATCH, LTOT), jnp.bool_),
        ],
        compiler_params=pltpu.CompilerParams(
            dimension_semantics=("arbitrary",),
        ),
    )(lengths1, b1_flat, b2_flat, l1lane, l1c, l2c, wbd)
    return out, mask
